# baseline (speedup 1.0000x reference)
import jax
import jax.numpy as jnp
from jax import lax
from jax.experimental import pallas as pl
from jax.experimental.pallas import tpu as pltpu
from jax.experimental.pallas import tpu_sc as plsc

D_MODEL = 1024
SEQ = 4096
BATCH = 4
NW = 32
NTILE = 16
SEQ_PER_W = SEQ // NW
CHUNK_ROWS = 16
N_CHUNKS = SEQ_PER_W // CHUNK_ROWS
N_UNITS = N_CHUNKS * BATCH
LANES = 16
NBUF = 4
NS1 = 2


def _sc_body(x_hbm, pos_hbm, out_hbm,
             shared, pbuf0, pbuf1, xbuf0, xbuf1, xbuf2, xbuf3,
             psem0, psem1,
             asem0, asem1, asem2, asem3,
             isem0, isem1, isem2, isem3,
             osem0, osem1, osem2, osem3):
    cid = lax.axis_index("c")
    sid = lax.axis_index("s")
    wid = sid * 2 + cid
    seq_base = wid * SEQ_PER_W

    pbufs = (pbuf0, pbuf1)
    xbufs = (xbuf0, xbuf1, xbuf2, xbuf3)
    psems = (psem0, psem1)
    asems = (asem0, asem1, asem2, asem3)
    isems = (isem0, isem1, isem2, isem3)
    osems = (osem0, osem1, osem2, osem3)

    def row0_of(c):
        return seq_base + c * CHUNK_ROWS

    def start_pos(c):
        return pltpu.async_copy(
            pos_hbm.at[pl.ds(row0_of(c), CHUNK_ROWS)],
            pbufs[c % 2], psems[c % 2])

    def start_in1(u):
        c, b = u // 4, u % 4
        return pltpu.async_copy(
            x_hbm.at[b, pl.ds(row0_of(c), CHUNK_ROWS)],
            shared.at[sid, u % NS1], asems[u % NS1])

    def start_in2(u):
        return pltpu.async_copy(
            shared.at[sid, u % NS1], xbufs[u % NBUF], isems[u % NBUF])

    def start_out(u):
        c, b = u // 4, u % 4
        return pltpu.async_copy(
            xbufs[u % NBUF],
            out_hbm.at[b, pl.ds(row0_of(c), CHUNK_ROWS)],
            osems[u % NBUF])

    def add_unit(u):
        buf = xbufs[u % NBUF]
        pb = pbufs[(u // 4) % 2]

        def blk_step(i, c2):
            r = lax.shift_right_logical(i, 3)
            off = pl.multiple_of(
                lax.shift_left(lax.bitwise_and(i, 7), 7), 128)
            vals = [pb[r, pl.ds(off + j * LANES, LANES)] for j in range(8)]
            for j in range(8):
                plsc.addupdate(buf.at[r, pl.ds(off + j * LANES, LANES)],
                               vals[j])
            return c2
        lax.fori_loop(0, CHUNK_ROWS * 8, blk_step, 0)

    pos_copies = [None, None]
    in1_copies = [None] * NS1
    in2_copies = [None] * NBUF
    out_copies = [None] * NBUF

    pos_copies[0] = start_pos(0)
    pos_copies[1] = start_pos(1)
    in1_copies[0] = start_in1(0)
    in1_copies[1] = start_in1(1)
    in1_copies[0].wait()
    in2_copies[0] = start_in2(0)

    for u in range(N_UNITS):
        c = u // 4
        if u % 4 == 0:
            pos_copies[c % 2].wait()
        if u >= 2:
            out_copies[(u + 2) % NBUF].wait()
        in2_copies[u % NBUF].wait()
        if u + 2 < N_UNITS:
            in1_copies[(u + 2) % NS1] = start_in1(u + 2)
        if u + 1 < N_UNITS:
            in1_copies[(u + 1) % NS1].wait()
            in2_copies[(u + 1) % NBUF] = start_in2(u + 1)
        add_unit(u)
        out_copies[u % NBUF] = start_out(u)
        if u % 4 == 3 and (c + 2) < N_CHUNKS:
            pos_copies[(c + 2) % 2] = start_pos(c + 2)

    out_copies[(N_UNITS - 2) % NBUF].wait()
    out_copies[(N_UNITS - 1) % NBUF].wait()


@jax.jit
def _sc_add(x, pos_embedding):
    mesh = plsc.VectorSubcoreMesh(core_axis_name="c", subcore_axis_name="s")
    return pl.kernel(
        _sc_body,
        out_type=jax.ShapeDtypeStruct((BATCH, SEQ, D_MODEL), jnp.float32),
        mesh=mesh,
        scratch_types=[
            pltpu.VMEM_SHARED((NTILE, NS1, CHUNK_ROWS, D_MODEL),
                              jnp.float32),
            pltpu.VMEM((CHUNK_ROWS, D_MODEL), jnp.float32),
            pltpu.VMEM((CHUNK_ROWS, D_MODEL), jnp.float32),
            pltpu.VMEM((CHUNK_ROWS, D_MODEL), jnp.float32),
            pltpu.VMEM((CHUNK_ROWS, D_MODEL), jnp.float32),
            pltpu.VMEM((CHUNK_ROWS, D_MODEL), jnp.float32),
            pltpu.VMEM((CHUNK_ROWS, D_MODEL), jnp.float32),
        ] + [pltpu.SemaphoreType.DMA] * 14,
    )(x, pos_embedding)


def kernel(x, pos_embedding):
    return _sc_add(x, pos_embedding)


# speedup vs baseline: 1.0425x; 1.0425x over previous
import jax
import jax.numpy as jnp
from jax import lax
from jax.experimental import pallas as pl
from jax.experimental.pallas import tpu as pltpu
from jax.experimental.pallas import tpu_sc as plsc

D_MODEL = 1024
SEQ = 4096
BATCH = 4
NW = 32
SEQ_PER_W = SEQ // NW
CHUNK_ROWS = 16
N_CHUNKS = SEQ_PER_W // CHUNK_ROWS
N_UNITS = N_CHUNKS * BATCH
LANES = 16
SLICES_PER_ROW = D_MODEL // LANES
NBUF = 4


def _sc_body(x_hbm, pos_hbm, out_hbm,
             pbuf0, pbuf1, xbuf0, xbuf1, xbuf2, xbuf3,
             psem0, psem1, isem0, isem1, isem2, isem3,
             osem0, osem1, osem2, osem3):
    cid = lax.axis_index("c")
    sid = lax.axis_index("s")
    wid = sid * 2 + cid
    seq_base = wid * SEQ_PER_W

    pbufs = (pbuf0, pbuf1)
    xbufs = (xbuf0, xbuf1, xbuf2, xbuf3)
    psems = (psem0, psem1)
    isems = (isem0, isem1, isem2, isem3)
    osems = (osem0, osem1, osem2, osem3)

    def row0_of(c):
        return seq_base + c * CHUNK_ROWS

    def start_pos(c):
        return pltpu.async_copy(
            pos_hbm.at[pl.ds(row0_of(c), CHUNK_ROWS)],
            pbufs[c % 2], psems[c % 2])

    def start_in(u):
        c, b = u // 4, u % 4
        return pltpu.async_copy(
            x_hbm.at[b, pl.ds(row0_of(c), CHUNK_ROWS)],
            xbufs[u % NBUF], isems[u % NBUF])

    def start_out(u):
        c, b = u // 4, u % 4
        return pltpu.async_copy(
            xbufs[u % NBUF],
            out_hbm.at[b, pl.ds(row0_of(c), CHUNK_ROWS)],
            osems[u % NBUF])

    def add_unit(u):
        buf = xbufs[u % NBUF]
        pb = pbufs[(u // 4) % 2]

        def blk_step(i, c2):
            r = lax.shift_right_logical(i, 3)
            off = pl.multiple_of(
                lax.shift_left(lax.bitwise_and(i, 7), 7), 128)
            vals = [pb[r, pl.ds(off + j * LANES, LANES)] for j in range(8)]
            for j in range(8):
                plsc.addupdate(buf.at[r, pl.ds(off + j * LANES, LANES)],
                               vals[j])
            return c2
        lax.fori_loop(0, CHUNK_ROWS * 8, blk_step, 0)

    pos_copies = [None, None]
    in_copies = [None] * NBUF
    out_copies = [None] * NBUF

    pos_copies[0] = start_pos(0)
    pos_copies[1] = start_pos(1)
    in_copies[0] = start_in(0)
    in_copies[1] = start_in(1)

    for u in range(N_UNITS):
        c = u // 4
        if u % 4 == 0:
            pos_copies[c % 2].wait()
        if u >= 2:
            out_copies[(u + 2) % NBUF].wait()
        if u + 2 < N_UNITS:
            in_copies[(u + 2) % NBUF] = start_in(u + 2)
        in_copies[u % NBUF].wait()
        add_unit(u)
        out_copies[u % NBUF] = start_out(u)
        if u % 4 == 3 and (c + 2) < N_CHUNKS:
            pos_copies[(c + 2) % 2] = start_pos(c + 2)

    out_copies[(N_UNITS - 2) % NBUF].wait()
    out_copies[(N_UNITS - 1) % NBUF].wait()


@jax.jit
def _sc_add(x, pos_embedding):
    mesh = plsc.VectorSubcoreMesh(core_axis_name="c", subcore_axis_name="s")
    return pl.kernel(
        _sc_body,
        out_type=jax.ShapeDtypeStruct((BATCH, SEQ, D_MODEL), jnp.float32),
        mesh=mesh,
        scratch_types=[
            pltpu.VMEM((CHUNK_ROWS, D_MODEL), jnp.float32),
            pltpu.VMEM((CHUNK_ROWS, D_MODEL), jnp.float32),
            pltpu.VMEM((CHUNK_ROWS, D_MODEL), jnp.float32),
            pltpu.VMEM((CHUNK_ROWS, D_MODEL), jnp.float32),
            pltpu.VMEM((CHUNK_ROWS, D_MODEL), jnp.float32),
            pltpu.VMEM((CHUNK_ROWS, D_MODEL), jnp.float32),
            pltpu.SemaphoreType.DMA,
            pltpu.SemaphoreType.DMA,
            pltpu.SemaphoreType.DMA,
            pltpu.SemaphoreType.DMA,
            pltpu.SemaphoreType.DMA,
            pltpu.SemaphoreType.DMA,
            pltpu.SemaphoreType.DMA,
            pltpu.SemaphoreType.DMA,
            pltpu.SemaphoreType.DMA,
            pltpu.SemaphoreType.DMA,
        ],
    )(x, pos_embedding)


def kernel(x, pos_embedding):
    return _sc_add(x, pos_embedding)
